# baseline (speedup 1.0000x reference)
import functools

import jax
import jax.numpy as jnp
from jax import lax
from jax.experimental import pallas as pl
from jax.experimental.pallas import tpu as pltpu
from jax.experimental.pallas import tpu_sc as plsc

N = 10000
E = 320000
D_IN = 128
D_H1 = 100
D_H2 = 200
D_OUT = 128

NC, NS = 2, 16
NW = NC * NS
NP = 10240
DP = 128
CHUNK = 128
EPT = 10240
NCHUNK = EPT // CHUNK
E_PAD = EPT * NW
STRIPE = NP // NS


@functools.lru_cache(maxsize=None)
def _sc_mesh():
    return plsc.VectorSubcoreMesh(
        core_axis_name="c", subcore_axis_name="s",
        num_cores=NC, num_subcores=NS)


_DEG_UN = 8


def _deg_body(dst_hbm, zeros_hbm, out_hbm, dst_all, ones_buf, cnt_sh, sem):
    cid = lax.axis_index("c")
    sid = lax.axis_index("s")
    tb = cid * NS + sid

    def fill_ones(i, _):
        ones_buf[i] = jnp.full((16,), 1.0, jnp.float32)
        return 0
    lax.fori_loop(0, CHUNK, fill_ones, 0)

    pltpu.sync_copy(dst_hbm.at[tb], dst_all)
    pltpu.sync_copy(zeros_hbm.at[pl.ds(sid * STRIPE, STRIPE)],
                    cnt_sh.at[pl.ds(sid * STRIPE, STRIPE)])
    plsc.subcore_barrier()

    def rnd(i, _):
        for k in range(_DEG_UN):
            pltpu.async_copy(ones_buf, cnt_sh.at[dst_all.at[i * _DEG_UN + k]],
                             sem, add=True)
        for k in range(_DEG_UN):
            pltpu.make_async_copy(
                ones_buf, cnt_sh.at[dst_all.at[i * _DEG_UN + k]], sem).wait()
        return 0
    lax.fori_loop(0, NCHUNK // _DEG_UN, rnd, 0)

    plsc.subcore_barrier()
    pltpu.sync_copy(cnt_sh.at[pl.ds(sid * STRIPE, STRIPE)],
                    out_hbm.at[cid, pl.ds(sid * STRIPE, STRIPE)])


@functools.lru_cache(maxsize=None)
def _deg_kernel_fn():
    return pl.kernel(
        _deg_body,
        out_type=jax.ShapeDtypeStruct((NC, NP, 16), jnp.float32),
        mesh=_sc_mesh(),
        scratch_types=[
            pltpu.VMEM((NCHUNK, CHUNK), jnp.int32),
            pltpu.VMEM((CHUNK, 16), jnp.float32),
            pltpu.VMEM_SHARED((NP, 16), jnp.float32),
            pltpu.SemaphoreType.DMA,
        ],
    )


def _deg_kernel(dst, zeros16):
    return _deg_kernel_fn()(dst, zeros16)


TOTCH = E_PAD // CHUNK
C0 = 40
C1 = TOTCH // NS - C0
CMAX = max(C0, C1)


def _edge_stream(nch, base, g_hbm, src_hbm, dst_hbm,
                 src_all, dstbuf, rows, acc_sh, gsems, dsems):
    pltpu.sync_copy(src_hbm.at[pl.ds(base * CHUNK, nch * CHUNK)],
                    src_all.at[pl.ds(0, nch * CHUNK)])

    def fire(c, k):
        pltpu.async_copy(g_hbm.at[src_all.at[pl.ds(c * CHUNK, CHUNK)]],
                         rows.at[k], gsems.at[k])
        pltpu.async_copy(dst_hbm.at[base + c], dstbuf.at[k], dsems.at[k])

    def consume(c, k):
        pltpu.make_async_copy(g_hbm.at[src_all.at[pl.ds(c * CHUNK, CHUNK)]],
                              rows.at[k], gsems.at[k]).wait()
        pltpu.make_async_copy(dst_hbm.at[base + c], dstbuf.at[k],
                              dsems.at[k]).wait()
        pltpu.sync_copy(rows.at[k], acc_sh.at[dstbuf.at[k]], add=True)

    fire(0, 0)

    def body(i, _):
        for k in range(2):
            c = i * 2 + k

            @pl.when(c + 1 < nch)
            def _():
                fire(c + 1, (k + 1) % 2)

            consume(c, k)
        return 0
    lax.fori_loop(0, nch // 2, body, 0)


def _edge_body(g_hbm, src_hbm, dst_hbm, zeros_hbm, out_hbm,
               src_all, dstbuf, rows, acc_sh, gsems, dsems):
    cid = lax.axis_index("c")
    sid = lax.axis_index("s")

    pltpu.sync_copy(zeros_hbm.at[pl.ds(sid * STRIPE, STRIPE)],
                    acc_sh.at[pl.ds(sid * STRIPE, STRIPE)])
    plsc.subcore_barrier()

    @pl.when(cid == 0)
    def _():
        _edge_stream(C0, sid * C0, g_hbm, src_hbm, dst_hbm,
                     src_all, dstbuf, rows, acc_sh, gsems, dsems)

    @pl.when(cid == 1)
    def _():
        _edge_stream(C1, NS * C0 + sid * C1, g_hbm, src_hbm, dst_hbm,
                     src_all, dstbuf, rows, acc_sh, gsems, dsems)

    plsc.subcore_barrier()
    pltpu.sync_copy(acc_sh.at[pl.ds(sid * STRIPE, STRIPE)],
                    out_hbm.at[cid, pl.ds(sid * STRIPE, STRIPE)])


@functools.lru_cache(maxsize=None)
def _edge_kernel_fn():
    return pl.kernel(
        _edge_body,
        out_type=jax.ShapeDtypeStruct((NC, NP, DP), jnp.float32),
        mesh=_sc_mesh(),
        scratch_types=[
            pltpu.VMEM((CMAX * CHUNK,), jnp.int32),
            pltpu.VMEM((2, CHUNK), jnp.int32),
            pltpu.VMEM((2, CHUNK, DP), jnp.float32),
            pltpu.VMEM_SHARED((NP, DP), jnp.float32),
            pltpu.SemaphoreType.DMA((2,)),
            pltpu.SemaphoreType.DMA((2,)),
        ],
    )


def _edge_kernel(g, src, dst, zerosDP):
    return _edge_kernel_fn()(g, src, dst, zerosDP)


_RB = 1024


def _dinv_blk(cnt0, cnt1):
    deg = cnt0[:, :1] + cnt1[:, :1] + 1.0
    return lax.rsqrt(deg)


def _dense1_body(x_ref, w1_ref, cnt0_ref, cnt1_ref, h1_ref, g1_ref):
    h1 = jnp.dot(x_ref[...], w1_ref[...], preferred_element_type=jnp.float32)
    dinv = _dinv_blk(cnt0_ref[...], cnt1_ref[...])
    h1_ref[...] = h1
    g1_ref[...] = h1 * dinv


def _mid_body(s0_ref, s1_ref, h1_ref, cnt0_ref, cnt1_ref, b1_ref,
              x2_ref, g2_ref):
    dinv = _dinv_blk(cnt0_ref[...], cnt1_ref[...])
    s = s0_ref[...] + s1_ref[...]
    x2 = jnp.maximum(dinv * s + (dinv * dinv) * h1_ref[...] + b1_ref[...], 0.0)
    x2_ref[...] = x2
    g2_ref[...] = x2 * dinv


def _final_body(s0_ref, s1_ref, x2_ref, cnt0_ref, cnt1_ref,
                w2_ref, b2_ref, wfc_ref, bfc_ref, out_ref):
    dinv = _dinv_blk(cnt0_ref[...], cnt1_ref[...])
    ax2 = dinv * (s0_ref[...] + s1_ref[...]) + (dinv * dinv) * x2_ref[...]
    t = jnp.dot(ax2, w2_ref[...], preferred_element_type=jnp.float32)
    t = jnp.maximum(t + b2_ref[...], 0.0)
    out_ref[...] = jnp.dot(t, wfc_ref[...],
                           preferred_element_type=jnp.float32) + bfc_ref[...]


def _row_spec(d):
    return pl.BlockSpec((_RB, d), lambda i: (i, 0))


def _full_spec(r, c):
    return pl.BlockSpec((r, c), lambda i: (0, 0))


def kernel(edge_features, edge_indices, W1, b1, W2, b2, Wfc, bfc):
    f32 = jnp.float32
    ei = edge_indices.astype(jnp.int32)
    pad_e = E_PAD - E
    src = jnp.concatenate(
        [ei[0], jnp.full((pad_e + CMAX * CHUNK,), N, jnp.int32)])
    dst = jnp.concatenate([ei[1], jnp.full((pad_e,), N, jnp.int32)])
    dst_e = dst.reshape(TOTCH, CHUNK)
    dst_deg = dst.reshape(NW, NCHUNK, CHUNK)

    x = jnp.pad(edge_features.astype(f32), ((0, NP - N), (0, 0)))
    w1p = jnp.pad(W1.astype(f32), ((0, 0), (0, DP - D_H1)))
    b1p = jnp.pad(b1.astype(f32), (0, DP - D_H1)).reshape(1, DP)
    w2p = jnp.pad(W2.astype(f32), ((0, DP - D_H1), (0, 256 - D_H2)))
    b2p = jnp.pad(b2.astype(f32), (0, 256 - D_H2)).reshape(1, 256)
    wfcp = jnp.pad(Wfc.astype(f32), ((0, 256 - D_H2), (0, 0)))
    bfcp = bfc.astype(f32).reshape(1, D_OUT)

    zeros16 = jnp.zeros((NP, 16), f32)
    zerosDP = jnp.zeros((NP, DP), f32)

    cnt = _deg_kernel(dst_deg, zeros16)
    cnt0, cnt1 = cnt[0], cnt[1]

    grid = NP // _RB
    h1, g1 = pl.pallas_call(
        _dense1_body,
        grid=(grid,),
        in_specs=[_row_spec(D_IN), _full_spec(D_IN, DP),
                  _row_spec(16), _row_spec(16)],
        out_specs=[_row_spec(DP), _row_spec(DP)],
        out_shape=[jax.ShapeDtypeStruct((NP, DP), f32)] * 2,
    )(x, w1p, cnt0, cnt1)

    s1 = _edge_kernel(g1, src, dst_e, zerosDP)

    x2, g2 = pl.pallas_call(
        _mid_body,
        grid=(grid,),
        in_specs=[_row_spec(DP), _row_spec(DP), _row_spec(DP),
                  _row_spec(16), _row_spec(16), _full_spec(1, DP)],
        out_specs=[_row_spec(DP), _row_spec(DP)],
        out_shape=[jax.ShapeDtypeStruct((NP, DP), f32)] * 2,
    )(s1[0], s1[1], h1, cnt0, cnt1, b1p)

    s2 = _edge_kernel(g2, src, dst_e, zerosDP)

    out = pl.pallas_call(
        _final_body,
        grid=(grid,),
        in_specs=[_row_spec(DP), _row_spec(DP), _row_spec(DP),
                  _row_spec(16), _row_spec(16),
                  _full_spec(DP, 256), _full_spec(1, 256),
                  _full_spec(256, D_OUT), _full_spec(1, D_OUT)],
        out_specs=_row_spec(D_OUT),
        out_shape=jax.ShapeDtypeStruct((NP, D_OUT), f32),
    )(s2[0], s2[1], x2, cnt0, cnt1, w2p, b2p, wfcp, bfcp)

    return out[:N]


# speedup vs baseline: 1.0430x; 1.0430x over previous
import functools

import jax
import jax.numpy as jnp
from jax import lax
from jax.experimental import pallas as pl
from jax.experimental.pallas import tpu as pltpu
from jax.experimental.pallas import tpu_sc as plsc

N = 10000
E = 320000
D_IN = 128
D_H1 = 100
D_H2 = 200
D_OUT = 128

NC, NS = 2, 16
NW = NC * NS
NP = 10240
DP = 128
CHUNK = 128
EPT = 10240
NCHUNK = EPT // CHUNK
E_PAD = EPT * NW
STRIPE = NP // NS


@functools.lru_cache(maxsize=None)
def _sc_mesh():
    return plsc.VectorSubcoreMesh(
        core_axis_name="c", subcore_axis_name="s",
        num_cores=NC, num_subcores=NS)


_DEG_UN = 8


def _deg_body(dst_hbm, zeros_hbm, out_hbm, dst_all, ones_buf, cnt_sh, sem):
    cid = lax.axis_index("c")
    sid = lax.axis_index("s")
    tb = cid * NS + sid

    def fill_ones(i, _):
        ones_buf[i] = jnp.full((16,), 1.0, jnp.float32)
        return 0
    lax.fori_loop(0, CHUNK, fill_ones, 0)

    pltpu.sync_copy(dst_hbm.at[tb], dst_all)
    pltpu.sync_copy(zeros_hbm.at[pl.ds(sid * STRIPE, STRIPE)],
                    cnt_sh.at[pl.ds(sid * STRIPE, STRIPE)])
    plsc.subcore_barrier()

    def rnd(i, _):
        for k in range(_DEG_UN):
            pltpu.async_copy(ones_buf, cnt_sh.at[dst_all.at[i * _DEG_UN + k]],
                             sem, add=True)
        for k in range(_DEG_UN):
            pltpu.make_async_copy(
                ones_buf, cnt_sh.at[dst_all.at[i * _DEG_UN + k]], sem).wait()
        return 0
    lax.fori_loop(0, NCHUNK // _DEG_UN, rnd, 0)

    plsc.subcore_barrier()
    pltpu.sync_copy(cnt_sh.at[pl.ds(sid * STRIPE, STRIPE)],
                    out_hbm.at[cid, pl.ds(sid * STRIPE, STRIPE)])


@functools.lru_cache(maxsize=None)
def _deg_kernel_fn():
    return pl.kernel(
        _deg_body,
        out_type=jax.ShapeDtypeStruct((NC, NP, 16), jnp.float32),
        mesh=_sc_mesh(),
        scratch_types=[
            pltpu.VMEM((NCHUNK, CHUNK), jnp.int32),
            pltpu.VMEM((CHUNK, 16), jnp.float32),
            pltpu.VMEM_SHARED((NP, 16), jnp.float32),
            pltpu.SemaphoreType.DMA,
        ],
    )


def _deg_kernel(dst, zeros16):
    return _deg_kernel_fn()(dst, zeros16)


TOTCH = E_PAD // CHUNK
C0 = 80
C1 = TOTCH // NS - C0
CMAX = max(C0, C1)


def _edge_stream(nch, base, g_hbm, src_hbm, dst_hbm,
                 src_all, dstbuf, rows, acc_sh, gsems, dsems):
    pltpu.sync_copy(src_hbm.at[pl.ds(base * CHUNK, nch * CHUNK)],
                    src_all.at[pl.ds(0, nch * CHUNK)])

    def fire(c, k):
        pltpu.async_copy(g_hbm.at[src_all.at[pl.ds(c * CHUNK, CHUNK)]],
                         rows.at[k], gsems.at[k])
        pltpu.async_copy(dst_hbm.at[base + c], dstbuf.at[k], dsems.at[k])

    def consume(c, k):
        pltpu.make_async_copy(g_hbm.at[src_all.at[pl.ds(c * CHUNK, CHUNK)]],
                              rows.at[k], gsems.at[k]).wait()
        pltpu.make_async_copy(dst_hbm.at[base + c], dstbuf.at[k],
                              dsems.at[k]).wait()
        pltpu.sync_copy(rows.at[k], acc_sh.at[dstbuf.at[k]], add=True)

    fire(0, 0)

    def body(i, _):
        for k in range(2):
            c = i * 2 + k

            @pl.when(c + 1 < nch)
            def _():
                fire(c + 1, (k + 1) % 2)

            consume(c, k)
        return 0
    lax.fori_loop(0, nch // 2, body, 0)


def _edge_body(g_hbm, src_hbm, dst_hbm, zeros_hbm, out_hbm,
               src_all, dstbuf, rows, acc_sh, gsems, dsems):
    cid = lax.axis_index("c")
    sid = lax.axis_index("s")

    pltpu.sync_copy(zeros_hbm.at[pl.ds(sid * STRIPE, STRIPE)],
                    acc_sh.at[pl.ds(sid * STRIPE, STRIPE)])
    plsc.subcore_barrier()

    @pl.when(cid == 0)
    def _():
        _edge_stream(C0, sid * C0, g_hbm, src_hbm, dst_hbm,
                     src_all, dstbuf, rows, acc_sh, gsems, dsems)

    @pl.when(cid == 1)
    def _():
        _edge_stream(C1, NS * C0 + sid * C1, g_hbm, src_hbm, dst_hbm,
                     src_all, dstbuf, rows, acc_sh, gsems, dsems)

    plsc.subcore_barrier()
    pltpu.sync_copy(acc_sh.at[pl.ds(sid * STRIPE, STRIPE)],
                    out_hbm.at[cid, pl.ds(sid * STRIPE, STRIPE)])


@functools.lru_cache(maxsize=None)
def _edge_kernel_fn():
    return pl.kernel(
        _edge_body,
        out_type=jax.ShapeDtypeStruct((NC, NP, DP), jnp.float32),
        mesh=_sc_mesh(),
        scratch_types=[
            pltpu.VMEM((CMAX * CHUNK,), jnp.int32),
            pltpu.VMEM((2, CHUNK), jnp.int32),
            pltpu.VMEM((2, CHUNK, DP), jnp.float32),
            pltpu.VMEM_SHARED((NP, DP), jnp.float32),
            pltpu.SemaphoreType.DMA((2,)),
            pltpu.SemaphoreType.DMA((2,)),
        ],
    )


def _edge_kernel(g, src, dst, zerosDP):
    return _edge_kernel_fn()(g, src, dst, zerosDP)


_RB = 1024


def _dinv_blk(cnt0, cnt1):
    deg = cnt0[:, :1] + cnt1[:, :1] + 1.0
    return lax.rsqrt(deg)


def _dense1_body(x_ref, w1_ref, cnt0_ref, cnt1_ref, h1_ref, g1_ref):
    h1 = jnp.dot(x_ref[...], w1_ref[...], preferred_element_type=jnp.float32)
    dinv = _dinv_blk(cnt0_ref[...], cnt1_ref[...])
    h1_ref[...] = h1
    g1_ref[...] = h1 * dinv


def _mid_body(s0_ref, s1_ref, h1_ref, cnt0_ref, cnt1_ref, b1_ref,
              x2_ref, g2_ref):
    dinv = _dinv_blk(cnt0_ref[...], cnt1_ref[...])
    s = s0_ref[...] + s1_ref[...]
    x2 = jnp.maximum(dinv * s + (dinv * dinv) * h1_ref[...] + b1_ref[...], 0.0)
    x2_ref[...] = x2
    g2_ref[...] = x2 * dinv


def _final_body(s0_ref, s1_ref, x2_ref, cnt0_ref, cnt1_ref,
                w2_ref, b2_ref, wfc_ref, bfc_ref, out_ref):
    dinv = _dinv_blk(cnt0_ref[...], cnt1_ref[...])
    ax2 = dinv * (s0_ref[...] + s1_ref[...]) + (dinv * dinv) * x2_ref[...]
    t = jnp.dot(ax2, w2_ref[...], preferred_element_type=jnp.float32)
    t = jnp.maximum(t + b2_ref[...], 0.0)
    out_ref[...] = jnp.dot(t, wfc_ref[...],
                           preferred_element_type=jnp.float32) + bfc_ref[...]


def _row_spec(d):
    return pl.BlockSpec((_RB, d), lambda i: (i, 0))


def _full_spec(r, c):
    return pl.BlockSpec((r, c), lambda i: (0, 0))


def kernel(edge_features, edge_indices, W1, b1, W2, b2, Wfc, bfc):
    f32 = jnp.float32
    ei = edge_indices.astype(jnp.int32)
    pad_e = E_PAD - E
    src = jnp.concatenate(
        [ei[0], jnp.full((pad_e + CMAX * CHUNK,), N, jnp.int32)])
    dst = jnp.concatenate([ei[1], jnp.full((pad_e,), N, jnp.int32)])
    dst_e = dst.reshape(TOTCH, CHUNK)
    dst_deg = dst.reshape(NW, NCHUNK, CHUNK)

    x = jnp.pad(edge_features.astype(f32), ((0, NP - N), (0, 0)))
    w1p = jnp.pad(W1.astype(f32), ((0, 0), (0, DP - D_H1)))
    b1p = jnp.pad(b1.astype(f32), (0, DP - D_H1)).reshape(1, DP)
    w2p = jnp.pad(W2.astype(f32), ((0, DP - D_H1), (0, 256 - D_H2)))
    b2p = jnp.pad(b2.astype(f32), (0, 256 - D_H2)).reshape(1, 256)
    wfcp = jnp.pad(Wfc.astype(f32), ((0, 256 - D_H2), (0, 0)))
    bfcp = bfc.astype(f32).reshape(1, D_OUT)

    zeros16 = jnp.zeros((NP, 16), f32)
    zerosDP = jnp.zeros((NP, DP), f32)

    cnt = _deg_kernel(dst_deg, zeros16)
    cnt0, cnt1 = cnt[0], cnt[1]

    grid = NP // _RB
    h1, g1 = pl.pallas_call(
        _dense1_body,
        grid=(grid,),
        in_specs=[_row_spec(D_IN), _full_spec(D_IN, DP),
                  _row_spec(16), _row_spec(16)],
        out_specs=[_row_spec(DP), _row_spec(DP)],
        out_shape=[jax.ShapeDtypeStruct((NP, DP), f32)] * 2,
    )(x, w1p, cnt0, cnt1)

    s1 = _edge_kernel(g1, src, dst_e, zerosDP)

    x2, g2 = pl.pallas_call(
        _mid_body,
        grid=(grid,),
        in_specs=[_row_spec(DP), _row_spec(DP), _row_spec(DP),
                  _row_spec(16), _row_spec(16), _full_spec(1, DP)],
        out_specs=[_row_spec(DP), _row_spec(DP)],
        out_shape=[jax.ShapeDtypeStruct((NP, DP), f32)] * 2,
    )(s1[0], s1[1], h1, cnt0, cnt1, b1p)

    s2 = _edge_kernel(g2, src, dst_e, zerosDP)

    out = pl.pallas_call(
        _final_body,
        grid=(grid,),
        in_specs=[_row_spec(DP), _row_spec(DP), _row_spec(DP),
                  _row_spec(16), _row_spec(16),
                  _full_spec(DP, 256), _full_spec(1, 256),
                  _full_spec(256, D_OUT), _full_spec(1, D_OUT)],
        out_specs=_row_spec(D_OUT),
        out_shape=jax.ShapeDtypeStruct((NP, D_OUT), f32),
    )(s2[0], s2[1], x2, cnt0, cnt1, w2p, b2p, wfcp, bfcp)

    return out[:N]


# speedup vs baseline: 1.4314x; 1.3724x over previous
import functools

import jax
import jax.numpy as jnp
from jax import lax
from jax.experimental import pallas as pl
from jax.experimental.pallas import tpu as pltpu
from jax.experimental.pallas import tpu_sc as plsc

N = 10000
E = 320000
D_IN = 128
D_H1 = 100
D_H2 = 200
D_OUT = 128

NC, NS = 2, 16
NW = NC * NS
NP = 10240
DP = 128
CHUNK = 128
EPT = 10240
NCHUNK = EPT // CHUNK
E_PAD = EPT * NW
STRIPE = NP // NS


@functools.lru_cache(maxsize=None)
def _sc_mesh():
    return plsc.VectorSubcoreMesh(
        core_axis_name="c", subcore_axis_name="s",
        num_cores=NC, num_subcores=NS)


_DEG_UN = 8


def _deg_body(dst_hbm, zeros_hbm, out_hbm, dst_all, ones_buf, cnt_sh, sem):
    cid = lax.axis_index("c")
    sid = lax.axis_index("s")
    tb = cid * NS + sid

    def fill_ones(i, _):
        ones_buf[i] = jnp.full((16,), 1.0, jnp.float32)
        return 0
    lax.fori_loop(0, CHUNK, fill_ones, 0)

    pltpu.sync_copy(dst_hbm.at[tb], dst_all)
    pltpu.sync_copy(zeros_hbm.at[pl.ds(sid * STRIPE, STRIPE)],
                    cnt_sh.at[pl.ds(sid * STRIPE, STRIPE)])
    plsc.subcore_barrier()

    def rnd(i, _):
        for k in range(_DEG_UN):
            pltpu.async_copy(ones_buf, cnt_sh.at[dst_all.at[i * _DEG_UN + k]],
                             sem, add=True)
        for k in range(_DEG_UN):
            pltpu.make_async_copy(
                ones_buf, cnt_sh.at[dst_all.at[i * _DEG_UN + k]], sem).wait()
        return 0
    lax.fori_loop(0, NCHUNK // _DEG_UN, rnd, 0)

    plsc.subcore_barrier()
    pltpu.sync_copy(cnt_sh.at[pl.ds(sid * STRIPE, STRIPE)],
                    out_hbm.at[cid, pl.ds(sid * STRIPE, STRIPE)])


@functools.lru_cache(maxsize=None)
def _deg_kernel_fn():
    return pl.kernel(
        _deg_body,
        out_type=jax.ShapeDtypeStruct((NC, NP, 16), jnp.float32),
        mesh=_sc_mesh(),
        scratch_types=[
            pltpu.VMEM((NCHUNK, CHUNK), jnp.int32),
            pltpu.VMEM((CHUNK, 16), jnp.float32),
            pltpu.VMEM_SHARED((NP, 16), jnp.float32),
            pltpu.SemaphoreType.DMA,
        ],
    )


def _deg_kernel(dst, zeros16):
    return _deg_kernel_fn()(dst, zeros16)


def _edge_body(g_hbm, src_hbm, dst_hbm, zeros_hbm, out_hbm,
               src_all, dstbuf, rows, acc_sh, gsems, dsems):
    cid = lax.axis_index("c")
    sid = lax.axis_index("s")
    tb = cid * NS + sid

    pltpu.sync_copy(src_hbm.at[tb], src_all)
    pltpu.sync_copy(zeros_hbm.at[pl.ds(sid * STRIPE, STRIPE)],
                    acc_sh.at[pl.ds(sid * STRIPE, STRIPE)])
    plsc.subcore_barrier()

    def fire(c, k):
        pltpu.async_copy(g_hbm.at[src_all.at[pl.ds(c * CHUNK, CHUNK)]],
                         rows.at[k], gsems.at[k])
        pltpu.async_copy(dst_hbm.at[tb, c], dstbuf.at[k], dsems.at[k])

    def consume(c, k):
        pltpu.make_async_copy(g_hbm.at[src_all.at[pl.ds(c * CHUNK, CHUNK)]],
                              rows.at[k], gsems.at[k]).wait()
        pltpu.make_async_copy(dst_hbm.at[tb, c], dstbuf.at[k],
                              dsems.at[k]).wait()
        pltpu.sync_copy(rows.at[k], acc_sh.at[dstbuf.at[k]], add=True)

    fire(0, 0)

    def body(i, _):
        for k in range(2):
            c = i * 2 + k

            @pl.when(c + 1 < NCHUNK)
            def _():
                fire(c + 1, (k + 1) % 2)

            consume(c, k)
        return 0
    lax.fori_loop(0, NCHUNK // 2, body, 0)

    plsc.subcore_barrier()
    pltpu.sync_copy(acc_sh.at[pl.ds(sid * STRIPE, STRIPE)],
                    out_hbm.at[cid, pl.ds(sid * STRIPE, STRIPE)])


@functools.lru_cache(maxsize=None)
def _edge_kernel_fn():
    return pl.kernel(
        _edge_body,
        out_type=jax.ShapeDtypeStruct((NC, NP, DP), jnp.float32),
        mesh=_sc_mesh(),
        scratch_types=[
            pltpu.VMEM((EPT,), jnp.int32),
            pltpu.VMEM((2, CHUNK), jnp.int32),
            pltpu.VMEM((2, CHUNK, DP), jnp.float32),
            pltpu.VMEM_SHARED((NP, DP), jnp.float32),
            pltpu.SemaphoreType.DMA((2,)),
            pltpu.SemaphoreType.DMA((2,)),
        ],
    )


def _edge_kernel(g, src, dst, zerosDP):
    return _edge_kernel_fn()(g, src, dst, zerosDP)


_RB = 1024


def _dinv_blk(cnt0, cnt1):
    deg = cnt0[:, :1] + cnt1[:, :1] + 1.0
    return lax.rsqrt(deg)


def _dense1_body(x_ref, w1_ref, cnt0_ref, cnt1_ref, h1_ref, g1_ref):
    h1 = jnp.dot(x_ref[...], w1_ref[...], preferred_element_type=jnp.float32)
    dinv = _dinv_blk(cnt0_ref[...], cnt1_ref[...])
    h1_ref[...] = h1
    g1_ref[...] = h1 * dinv


def _mid_body(s0_ref, s1_ref, h1_ref, cnt0_ref, cnt1_ref, b1_ref,
              x2_ref, g2_ref):
    dinv = _dinv_blk(cnt0_ref[...], cnt1_ref[...])
    s = s0_ref[...] + s1_ref[...]
    x2 = jnp.maximum(dinv * s + (dinv * dinv) * h1_ref[...] + b1_ref[...], 0.0)
    x2_ref[...] = x2
    g2_ref[...] = x2 * dinv


def _final_body(s0_ref, s1_ref, x2_ref, cnt0_ref, cnt1_ref,
                w2_ref, b2_ref, wfc_ref, bfc_ref, out_ref):
    dinv = _dinv_blk(cnt0_ref[...], cnt1_ref[...])
    ax2 = dinv * (s0_ref[...] + s1_ref[...]) + (dinv * dinv) * x2_ref[...]
    t = jnp.dot(ax2, w2_ref[...], preferred_element_type=jnp.float32)
    t = jnp.maximum(t + b2_ref[...], 0.0)
    out_ref[...] = jnp.dot(t, wfc_ref[...],
                           preferred_element_type=jnp.float32) + bfc_ref[...]


def _row_spec(d):
    return pl.BlockSpec((_RB, d), lambda i: (i, 0))


def _full_spec(r, c):
    return pl.BlockSpec((r, c), lambda i: (0, 0))


def kernel(edge_features, edge_indices, W1, b1, W2, b2, Wfc, bfc):
    f32 = jnp.float32
    ei = edge_indices.astype(jnp.int32)
    pad_e = E_PAD - E
    src = jnp.concatenate(
        [ei[0], jnp.full((pad_e,), N, jnp.int32)]).reshape(NW, EPT)
    dst = jnp.concatenate([ei[1], jnp.full((pad_e,), N, jnp.int32)])
    dst_e = dst.reshape(NW, NCHUNK, CHUNK)
    dst_deg = dst_e

    x = jnp.pad(edge_features.astype(f32), ((0, NP - N), (0, 0)))
    w1p = jnp.pad(W1.astype(f32), ((0, 0), (0, DP - D_H1)))
    b1p = jnp.pad(b1.astype(f32), (0, DP - D_H1)).reshape(1, DP)
    w2p = jnp.pad(W2.astype(f32), ((0, DP - D_H1), (0, 256 - D_H2)))
    b2p = jnp.pad(b2.astype(f32), (0, 256 - D_H2)).reshape(1, 256)
    wfcp = jnp.pad(Wfc.astype(f32), ((0, 256 - D_H2), (0, 0)))
    bfcp = bfc.astype(f32).reshape(1, D_OUT)

    zeros16 = jnp.zeros((NP, 16), f32)
    zerosDP = jnp.zeros((NP, DP), f32)

    cnt = _deg_kernel(dst_deg, zeros16)
    cnt0, cnt1 = cnt[0], cnt[1]

    grid = NP // _RB
    h1, g1 = pl.pallas_call(
        _dense1_body,
        grid=(grid,),
        in_specs=[_row_spec(D_IN), _full_spec(D_IN, DP),
                  _row_spec(16), _row_spec(16)],
        out_specs=[_row_spec(DP), _row_spec(DP)],
        out_shape=[jax.ShapeDtypeStruct((NP, DP), f32)] * 2,
    )(x, w1p, cnt0, cnt1)

    s1 = _edge_kernel(g1, src, dst_e, zerosDP)

    x2, g2 = pl.pallas_call(
        _mid_body,
        grid=(grid,),
        in_specs=[_row_spec(DP), _row_spec(DP), _row_spec(DP),
                  _row_spec(16), _row_spec(16), _full_spec(1, DP)],
        out_specs=[_row_spec(DP), _row_spec(DP)],
        out_shape=[jax.ShapeDtypeStruct((NP, DP), f32)] * 2,
    )(s1[0], s1[1], h1, cnt0, cnt1, b1p)

    s2 = _edge_kernel(g2, src, dst_e, zerosDP)

    out = pl.pallas_call(
        _final_body,
        grid=(grid,),
        in_specs=[_row_spec(DP), _row_spec(DP), _row_spec(DP),
                  _row_spec(16), _row_spec(16),
                  _full_spec(DP, 256), _full_spec(1, 256),
                  _full_spec(256, D_OUT), _full_spec(1, D_OUT)],
        out_specs=_row_spec(D_OUT),
        out_shape=jax.ShapeDtypeStruct((NP, D_OUT), f32),
    )(s2[0], s2[1], x2, cnt0, cnt1, w2p, b2p, wfcp, bfcp)

    return out[:N]
